# baseline (speedup 1.0000x reference)
import functools

import jax
import jax.numpy as jnp
from jax import lax
from jax.experimental import pallas as pl
from jax.experimental.pallas import tpu as pltpu
from jax.experimental.pallas import tpu_sc as plsc

ACTION_SIZE = 1000000
EMBED_DIM = 64

NC = 2
NS = 16
NW = NC * NS

B_TOTAL = 16384 * 50
B_PER_W = B_TOTAL // NW
BLK = 128
NBLK = B_PER_W // BLK
QUAD = 4
NBUF = 2 * QUAD
NQUAD = NBLK // QUAD


def _gather_body(idx_hbm, table_hbm, out_hbm, idx_v, rows_v, gsem, wsem):
    wid = lax.axis_index("s") * NC + lax.axis_index("c")
    base = wid * B_PER_W

    pltpu.sync_copy(idx_hbm.at[wid], idx_v)

    def gather_start(j, b):
        pltpu.make_async_copy(
            table_hbm.at[idx_v.at[j]], rows_v.at[b], gsem.at[b]
        ).start()

    def gather_wait(j, b):
        pltpu.make_async_copy(
            table_hbm.at[idx_v.at[j]], rows_v.at[b], gsem.at[b]
        ).wait()

    def write_start(j, b):
        pltpu.make_async_copy(
            rows_v.at[b], out_hbm.at[pl.ds(base + j * BLK, BLK)], wsem.at[b]
        ).start()

    def write_wait(j, b):
        pltpu.make_async_copy(
            rows_v.at[b], out_hbm.at[pl.ds(base + j * BLK, BLK)], wsem.at[b]
        ).wait()

    A = 0
    B = QUAD

    def quad_wait_write(q, s):
        for b in range(QUAD):
            gather_wait(q * QUAD + b, s + b)
            write_start(q * QUAD + b, s + b)

    def quad_rearm(q_done, q_next, s):
        for b in range(QUAD):
            write_wait(q_done * QUAD + b, s + b)
            gather_start(q_next * QUAD + b, s + b)

    for b in range(QUAD):
        gather_start(b, A + b)
    quad_wait_write(0, A)
    for b in range(QUAD):
        gather_start(QUAD + b, B + b)
    quad_wait_write(1, B)
    quad_rearm(0, 2, A)

    def pair_body(g, carry):
        qe = 2 * g
        quad_wait_write(qe, A)
        quad_rearm(qe - 1, qe + 1, B)
        quad_wait_write(qe + 1, B)
        quad_rearm(qe, qe + 2, A)
        return carry

    lax.fori_loop(1, NQUAD // 2 - 1, pair_body, 0)

    quad_wait_write(NQUAD - 2, A)
    quad_rearm(NQUAD - 3, NQUAD - 1, B)
    quad_wait_write(NQUAD - 1, B)
    for b in range(QUAD):
        write_wait((NQUAD - 2) * QUAD + b, A + b)
    for b in range(QUAD):
        write_wait((NQUAD - 1) * QUAD + b, B + b)


@jax.jit
def _embedding_gather(idx3, table):
    mesh = plsc.VectorSubcoreMesh(core_axis_name="c", subcore_axis_name="s")
    run = functools.partial(
        pl.kernel,
        out_type=jax.ShapeDtypeStruct((B_TOTAL, EMBED_DIM), jnp.float32),
        mesh=mesh,
        scratch_types=[
            pltpu.VMEM((NBLK, BLK), jnp.int32),
            pltpu.VMEM((NBUF, BLK, EMBED_DIM), jnp.float32),
            pltpu.SemaphoreType.DMA((NBUF,)),
            pltpu.SemaphoreType.DMA((NBUF,)),
        ],
        compiler_params=pltpu.CompilerParams(use_tc_tiling_on_sc=False),
    )(_gather_body)
    return run(idx3, table)


def kernel(action_indices, embedding_weight):
    idx3 = jnp.asarray(action_indices, jnp.int32).reshape(NW, NBLK, BLK)
    out = _embedding_gather(idx3, embedding_weight)
    return out.reshape(action_indices.shape + (EMBED_DIM,))
